# baseline (speedup 1.0000x reference)
import functools

import jax
import jax.numpy as jnp
from jax import lax
from jax.experimental import pallas as pl
from jax.experimental.pallas import tpu as pltpu

F32 = jnp.float32
CHUNK = 16


def _digin_block(vtT_ref, vpT_ref, adjT_ref, vsT_ref,
                 ttT_ref, ptT_ref, whT_ref, bhT_ref, epsv_ref,
                 wg1T_ref, bg1T_ref, wg2T_ref, bg2T_ref,
                 wp1Tr_ref, bp1T_ref, wp2T_ref, bp2T_ref,
                 ws1T_ref, bs1T_ref, ws2T_ref, bs2T_ref,
                 wgpgT_ref, wgpsT_ref, bgpT_ref, outT_ref):
    N, Bb = vtT_ref.shape
    EMB, NT = ttT_ref.shape
    NP = ptT_ref.shape[1]
    HID = wg1T_ref.shape[0]

    dot = functools.partial(jnp.dot, preferred_element_type=F32)

    one_eps = 1.0 + epsv_ref[0, 0]
    wtT = one_eps * dot(whT_ref[:, :EMB], ttT_ref[...])
    wpT = one_eps * dot(whT_ref[:, EMB:], ptT_ref[...])

    vtT = vtT_ref[...]
    vpT = vpT_ref[...]
    madj = adjT_ref[...]
    iota_t = lax.broadcasted_iota(jnp.int32, (NT, Bb), 0)
    iota_p = lax.broadcasted_iota(jnp.int32, (NP, Bb), 0)

    wg1T = wg1T_ref[...]
    bg1T = bg1T_ref[...]
    wg2T = wg2T_ref[...]
    bg2T = bg2T_ref[...]
    bhT = one_eps * bhT_ref[...]

    def hv_col(v):
        it = jnp.broadcast_to(vtT[v:v + 1, :], (HID, Bb))
        ip = jnp.broadcast_to(vpT[v:v + 1, :], (HID, Bb))
        return (jnp.take_along_axis(wtT, it, axis=1)
                + jnp.take_along_axis(wpT, ip, axis=1) + bhT)

    hs = []
    gaccT = jnp.zeros((bp1T_ref.shape[0], Bb), F32)
    for c in range(N // CHUNK):
        base = c * CHUNK
        accs = [hv_col(base + k) for k in range(CHUNK)]
        for u in range(base):
            hu = hs[u]
            for k in range(CHUNK):
                accs[k] = accs[k] + madj[base + k, u:u + 1, :] * hu
        for k in range(CHUNK):
            v = base + k
            x = accs[k]
            for j in range(k):
                x = x + madj[v, base + j:base + j + 1, :] * hs[base + j]
            hnew = dot(wg2T, jax.nn.relu(dot(wg1T, x) + bg1T)) + bg2T
            hs.append(hnew)
            gaccT = gaccT + dot(wp1Tr_ref[v], hnew)

    gT = dot(wp2T_ref[...], jax.nn.relu(gaccT + bp1T_ref[...])) + bp2T_ref[...]
    sT = dot(ws2T_ref[...],
             jax.nn.relu(dot(ws1T_ref[...], vsT_ref[...]) + bs1T_ref[...])
             ) + bs2T_ref[...]
    outT_ref[...] = (dot(wgpgT_ref[...], gT) + dot(wgpsT_ref[...], sT)
                     + bgpT_ref[...])


def kernel(v_types, v_paths, adj, v_sizes, type_table, path_table,
           Ws1, bs1, Ws2, bs2, Wh, bh, eps, Wg1, bg1, Wg2, bg2,
           Wp1, bp1, Wp2, bp2, Wgp, bgp):
    B, N = v_types.shape
    HID = Wg1.shape[0]
    LAT = Wgp.shape[1]
    P1 = Wp1.shape[1]
    Bb = 256 if B % 256 == 0 else B
    grid = (B // Bb,)

    def col(x):
        return x.reshape(-1, 1)

    vtT = v_types.T
    vpT = v_paths.T
    adjT = jnp.transpose(adj, (1, 2, 0)).astype(jnp.float32)
    vsT = v_sizes.T
    wp1Tr = jnp.transpose(Wp1.reshape(N, HID, P1), (0, 2, 1))

    weights = [type_table.T, path_table.T, Wh.T, col(bh), eps.reshape(1, 1),
               Wg1.T, col(bg1), Wg2.T, col(bg2),
               wp1Tr, col(bp1), Wp2.T, col(bp2),
               Ws1.T, col(bs1), Ws2.T, col(bs2),
               Wgp[:HID].T, Wgp[HID:].T, col(bgp)]

    data = [vtT, vpT, adjT, vsT]
    data_specs = [
        pl.BlockSpec((N, Bb), lambda i: (0, i)),
        pl.BlockSpec((N, Bb), lambda i: (0, i)),
        pl.BlockSpec((N, N, Bb), lambda i: (0, 0, i)),
        pl.BlockSpec((vsT.shape[0], Bb), lambda i: (0, i)),
    ]
    w_specs = [pl.BlockSpec(w.shape, lambda i, nd=w.ndim: (0,) * nd)
               for w in weights]

    outT = pl.pallas_call(
        _digin_block,
        grid=grid,
        in_specs=data_specs + w_specs,
        out_specs=pl.BlockSpec((LAT, Bb), lambda i: (0, i)),
        out_shape=jax.ShapeDtypeStruct((LAT, B), F32),
        compiler_params=pltpu.CompilerParams(
            dimension_semantics=("parallel",)),
    )(*data, *weights)
    return outT.T


# speedup vs baseline: 1.2237x; 1.2237x over previous
import functools

import jax
import jax.numpy as jnp
from jax import lax
from jax.experimental import pallas as pl
from jax.experimental.pallas import tpu as pltpu

F32 = jnp.float32
CHUNK = 16


def _digin_block(vtT_ref, vpT_ref, adjT_ref, vsT_ref,
                 ttT_ref, ptT_ref, whT_ref, bhT_ref, epsv_ref,
                 wg1T_ref, bg1T_ref, wg2T_ref, bg2T_ref,
                 wp1Tr_ref, bp1T_ref, wp2T_ref, bp2T_ref,
                 ws1T_ref, bs1T_ref, ws2T_ref, bs2T_ref,
                 wgpgT_ref, wgpsT_ref, bgpT_ref, outT_ref):
    N, Bb = vtT_ref.shape
    EMB, NT = ttT_ref.shape
    NP = ptT_ref.shape[1]
    HID = wg1T_ref.shape[0]

    dot = functools.partial(jnp.dot, preferred_element_type=F32)

    one_eps = 1.0 + epsv_ref[0, 0]
    wtT = one_eps * dot(whT_ref[:, :EMB], ttT_ref[...])
    wpT = one_eps * dot(whT_ref[:, EMB:], ptT_ref[...])

    vtT = vtT_ref[...]
    vpT = vpT_ref[...]
    madj = adjT_ref[...]
    iota_t = lax.broadcasted_iota(jnp.int32, (NT, Bb), 0)
    iota_p = lax.broadcasted_iota(jnp.int32, (NP, Bb), 0)

    wg1T = wg1T_ref[...]
    bg1T = bg1T_ref[...]
    wg2T = wg2T_ref[...]
    bg2T = bg2T_ref[...]
    bhT = one_eps * bhT_ref[...]

    def hv_col(v):
        it = jnp.broadcast_to(vtT[v:v + 1, :], (HID, Bb))
        ip = jnp.broadcast_to(vpT[v:v + 1, :], (HID, Bb))
        return (jnp.take_along_axis(wtT, it, axis=1)
                + jnp.take_along_axis(wpT, ip, axis=1) + bhT)

    hs = []
    gaccT = jnp.zeros((bp1T_ref.shape[0], Bb), F32)
    for c in range(N // CHUNK):
        base = c * CHUNK
        accs = [hv_col(base + k) for k in range(CHUNK)]
        for u in range(base):
            hu = hs[u]
            for k in range(CHUNK):
                accs[k] = accs[k] + madj[base + k, u:u + 1, :] * hu
        for k in range(CHUNK):
            v = base + k
            x = accs[k]
            for j in range(k):
                x = x + madj[v, base + j:base + j + 1, :] * hs[base + j]
            hnew = dot(wg2T, jax.nn.relu(dot(wg1T, x) + bg1T)) + bg2T
            hs.append(hnew)
            gaccT = gaccT + dot(wp1Tr_ref[v], hnew)

    gT = dot(wp2T_ref[...], jax.nn.relu(gaccT + bp1T_ref[...])) + bp2T_ref[...]
    sT = dot(ws2T_ref[...],
             jax.nn.relu(dot(ws1T_ref[...], vsT_ref[...]) + bs1T_ref[...])
             ) + bs2T_ref[...]
    outT_ref[...] = (dot(wgpgT_ref[...], gT) + dot(wgpsT_ref[...], sT)
                     + bgpT_ref[...])


def kernel(v_types, v_paths, adj, v_sizes, type_table, path_table,
           Ws1, bs1, Ws2, bs2, Wh, bh, eps, Wg1, bg1, Wg2, bg2,
           Wp1, bp1, Wp2, bp2, Wgp, bgp):
    B, N = v_types.shape
    HID = Wg1.shape[0]
    LAT = Wgp.shape[1]
    P1 = Wp1.shape[1]
    Bb = 512 if B % 512 == 0 else B
    grid = (B // Bb,)

    def col(x):
        return x.reshape(-1, 1)

    vtT = v_types.T
    vpT = v_paths.T
    adjT = jnp.transpose(adj, (1, 2, 0)).astype(jnp.float32)
    vsT = v_sizes.T
    wp1Tr = jnp.transpose(Wp1.reshape(N, HID, P1), (0, 2, 1))

    weights = [type_table.T, path_table.T, Wh.T, col(bh), eps.reshape(1, 1),
               Wg1.T, col(bg1), Wg2.T, col(bg2),
               wp1Tr, col(bp1), Wp2.T, col(bp2),
               Ws1.T, col(bs1), Ws2.T, col(bs2),
               Wgp[:HID].T, Wgp[HID:].T, col(bgp)]

    data = [vtT, vpT, adjT, vsT]
    data_specs = [
        pl.BlockSpec((N, Bb), lambda i: (0, i)),
        pl.BlockSpec((N, Bb), lambda i: (0, i)),
        pl.BlockSpec((N, N, Bb), lambda i: (0, 0, i)),
        pl.BlockSpec((vsT.shape[0], Bb), lambda i: (0, i)),
    ]
    w_specs = [pl.BlockSpec(w.shape, lambda i, nd=w.ndim: (0,) * nd)
               for w in weights]

    outT = pl.pallas_call(
        _digin_block,
        grid=grid,
        in_specs=data_specs + w_specs,
        out_specs=pl.BlockSpec((LAT, Bb), lambda i: (0, i)),
        out_shape=jax.ShapeDtypeStruct((LAT, B), F32),
        compiler_params=pltpu.CompilerParams(
            dimension_semantics=("parallel",)),
    )(*data, *weights)
    return outT.T
